# Initial kernel scaffold; baseline (speedup 1.0000x reference)
import jax
import jax.numpy as jnp
from jax.experimental import pallas as pl


def kernel(original_xyzs, dirs, static_table, tableA, tableB, table2A, table2B, W1, W2, C1, C2, C3):
    raise NotImplementedError("write your pallas kernel here")


# speedup vs baseline: 1.3277x; 1.3277x over previous
import functools

import numpy as np
import jax
import jax.numpy as jnp
from jax import lax
from jax.experimental import pallas as pl
from jax.experimental.pallas import tpu as pltpu
from jax.experimental.pallas import tpu_sc as plsc

_L = 16
_F = 2
_T = 2 ** 19
_NPTS = 65536
_B = float(np.exp(np.log(4096.0 / 16.0) / (_L - 1)))
_RES = [int(np.floor(16 * (_B ** l))) for l in range(_L)]
_P2I = int(np.uint32(2654435761).view(np.int32))
_P3I = int(np.uint32(805459861).view(np.int32))

_NW = 32
_PPW = _NPTS // _NW
_CPTS = 16
_NCH = _PPW // _CPTS
_D = 16
_OC = 112


def _sc_encode(xs, ys, zs, tbl, resv, parv):
    mesh = plsc.VectorSubcoreMesh(core_axis_name="c", subcore_axis_name="s")

    @functools.partial(
        pl.kernel,
        mesh=mesh,
        out_type=jax.ShapeDtypeStruct((_NPTS * _OC,), jnp.float32),
        compiler_params=pltpu.CompilerParams(
            needs_layout_passes=False, use_tc_tiling_on_sc=False),
        scratch_types=[
            pltpu.VMEM((_PPW,), jnp.float32),
            pltpu.VMEM((_PPW,), jnp.float32),
            pltpu.VMEM((_PPW,), jnp.float32),
            pltpu.VMEM((16,), jnp.float32),
            pltpu.VMEM((16,), jnp.float32),
            pltpu.VMEM((_L, 8 * _CPTS), jnp.int32),
            pltpu.VMEM((_L, 8 * _CPTS), jnp.float32),
            pltpu.VMEM((_L, 8 * _CPTS, _D), jnp.float32),
            pltpu.VMEM((_CPTS * _OC,), jnp.float32),
            pltpu.SemaphoreType.DMA,
        ],
    )
    def k(xs_h, ys_h, zs_h, tbl_h, res_h, par_h, out_h,
          x_s, y_s, z_s, res_s, par_s, idx_s, w_s, rows_s, stage_s, sem):
        wid = lax.axis_index("s") * 2 + lax.axis_index("c")
        base = wid * _PPW
        pltpu.sync_copy(xs_h.at[pl.ds(base, _PPW)], x_s)
        pltpu.sync_copy(ys_h.at[pl.ds(base, _PPW)], y_s)
        pltpu.sync_copy(zs_h.at[pl.ds(base, _PPW)], z_s)
        pltpu.sync_copy(res_h, res_s)
        pltpu.sync_copy(par_h, par_s)
        it = lax.iota(jnp.int32, 16)
        it_oc = it * _OC
        z16 = it * 0
        a1 = plsc.load_gather(par_s, [z16])
        b1 = plsc.load_gather(par_s, [z16 + 1])
        a2 = plsc.load_gather(par_s, [z16 + 2])
        b2 = plsc.load_gather(par_s, [z16 + 3])

        def chunk(kc, carry):
            po = kc * _CPTS
            x = x_s[pl.ds(po, _CPTS)] * 0.5 + 0.5
            y = y_s[pl.ds(po, _CPTS)] * 0.5 + 0.5
            z = z_s[pl.ds(po, _CPTS)] * 0.5 + 0.5

            def lvl_idx(l, c2):
                r = plsc.load_gather(res_s, [z16 + l])
                px = x * r
                py = y * r
                pz = z * r
                ix = px.astype(jnp.int32)
                iy = py.astype(jnp.int32)
                iz = pz.astype(jnp.int32)
                fx = px - ix.astype(jnp.float32)
                fy = py - iy.astype(jnp.float32)
                fz = pz - iz.astype(jnp.float32)
                gy = iy * _P2I
                gz = iz * _P3I
                lT = l * _T
                for c in range(8):
                    cx, cy, cz = c & 1, (c >> 1) & 1, (c >> 2) & 1
                    hx = ix + cx if cx else ix
                    hy = gy + _P2I if cy else gy
                    hz = gz + _P3I if cz else gz
                    h = ((hx ^ hy) ^ hz) & (_T - 1)
                    wx = fx if cx else 1.0 - fx
                    wy = fy if cy else 1.0 - fy
                    wz = fz if cz else 1.0 - fz
                    idx_s[l, pl.ds(c * _CPTS, _CPTS)] = h + lT
                    w_s[l, pl.ds(c * _CPTS, _CPTS)] = wx * wy * wz
                return c2

            lax.fori_loop(0, _L, lvl_idx, 0)

            copies = [
                pltpu.async_copy(tbl_h.at[idx_s.at[i]], rows_s.at[i], sem)
                for i in range(_L)
            ]
            for cp in copies:
                cp.wait()

            def lvl_acc(l, c2):
                lvec = z16 + l
                acc = [jnp.zeros((16,), jnp.float32) for _ in range(10)]
                for c in range(8):
                    w = w_s[l, pl.ds(c * _CPTS, _CPTS)]
                    rvec = c * _CPTS + it
                    for j in range(10):
                        cvec = z16 + j
                        v = plsc.load_gather(rows_s, [lvec, rvec, cvec])
                        acc[j] = acc[j] + w * v
                col = 2 * l
                plsc.store_scatter(stage_s, [it_oc + col], acc[0])
                plsc.store_scatter(stage_s, [it_oc + (col + 1)], acc[1])
                plsc.store_scatter(stage_s, [it_oc + (32 + col)], a1 * acc[2] + b1 * acc[4])
                plsc.store_scatter(stage_s, [it_oc + (33 + col)], a1 * acc[3] + b1 * acc[5])
                plsc.store_scatter(stage_s, [it_oc + (64 + col)], a2 * acc[6] + b2 * acc[8])
                plsc.store_scatter(stage_s, [it_oc + (65 + col)], a2 * acc[7] + b2 * acc[9])

                @pl.when(l >= 12)
                def _():
                    colA = 96 + 2 * (l - 12)
                    plsc.store_scatter(stage_s, [it_oc + colA], acc[2])
                    plsc.store_scatter(stage_s, [it_oc + (colA + 1)], acc[3])
                    plsc.store_scatter(stage_s, [it_oc + (colA + 8)], acc[4])
                    plsc.store_scatter(stage_s, [it_oc + (colA + 9)], acc[5])

                return c2

            lax.fori_loop(0, _L, lvl_acc, 0)

            pltpu.sync_copy(stage_s, out_h.at[pl.ds((base + po) * _OC, _CPTS * _OC)])
            return carry

        lax.fori_loop(0, _NCH, chunk, 0)

    return k(xs, ys, zs, tbl, resv, parv)


_BP = 2048


def _mlp_body(sc_ref, dirs_ref, w1_ref, w2_ref, c1_ref, c2_ref, c3_ref,
              sig_ref, col_ref):
    feat = sc_ref[:, :96]
    h1 = jnp.maximum(jnp.dot(feat, w1_ref[...], preferred_element_type=jnp.float32), 0.0)
    h = jnp.dot(h1, w2_ref[...], preferred_element_type=jnp.float32)
    sig_ref[...] = jnp.exp(h[:, 0:1])

    d = dirs_ref[...]
    inv = 1.0 / (jnp.sqrt(jnp.sum(d * d, axis=1, keepdims=True)) + 1e-8)
    dn = d * inv
    x = dn[:, 0:1]
    y = dn[:, 1:2]
    z = dn[:, 2:3]
    x2, y2, z2 = x * x, y * y, z * z
    xy, yz, xz = x * y, y * z, x * z
    comps = [
        0.28209479177387814 * jnp.ones_like(x),
        -0.48860251190291987 * y,
        0.48860251190291987 * z,
        -0.48860251190291987 * x,
        1.0925484305920792 * xy,
        -1.0925484305920792 * yz,
        0.94617469575755997 * z2 - 0.31539156525252005,
        -1.0925484305920792 * xz,
        0.54627421529603959 * (x2 - y2),
        -0.59004358992664352 * y * (3.0 * x2 - y2),
        2.8906114426405538 * xy * z,
        -0.45704579946446572 * y * (4.0 * z2 - x2 - y2),
        0.3731763325901154 * z * (2.0 * z2 - 3.0 * x2 - 3.0 * y2),
        -0.45704579946446572 * x * (4.0 * z2 - x2 - y2),
        1.4453057213202769 * z * (x2 - y2),
        -0.59004358992664352 * x * (x2 - 3.0 * y2),
    ]
    sh = jnp.concatenate(comps, axis=1)
    ci1 = (jnp.dot(sh, c1_ref[:16, :], preferred_element_type=jnp.float32)
           + jnp.dot(h, c1_ref[16:, :], preferred_element_type=jnp.float32))
    cc = jnp.maximum(ci1, 0.0)
    cc = jnp.maximum(jnp.dot(cc, c2_ref[...], preferred_element_type=jnp.float32), 0.0)
    col_ref[...] = jax.nn.sigmoid(jnp.dot(cc, c3_ref[...], preferred_element_type=jnp.float32))


def _tc_mlp(sc_out, dirs, W1, W2, C1, C2, C3):
    grid = (_NPTS // _BP,)
    return pl.pallas_call(
        _mlp_body,
        grid=grid,
        in_specs=[
            pl.BlockSpec((_BP, _OC), lambda i: (i, 0)),
            pl.BlockSpec((_BP, 3), lambda i: (i, 0)),
            pl.BlockSpec((96, 64), lambda i: (0, 0)),
            pl.BlockSpec((64, 16), lambda i: (0, 0)),
            pl.BlockSpec((32, 64), lambda i: (0, 0)),
            pl.BlockSpec((64, 64), lambda i: (0, 0)),
            pl.BlockSpec((64, 3), lambda i: (0, 0)),
        ],
        out_specs=[
            pl.BlockSpec((_BP, 1), lambda i: (i, 0)),
            pl.BlockSpec((_BP, 3), lambda i: (i, 0)),
        ],
        out_shape=[
            jax.ShapeDtypeStruct((_NPTS, 1), jnp.float32),
            jax.ShapeDtypeStruct((_NPTS, 3), jnp.float32),
        ],
    )(sc_out, dirs, W1, W2, C1, C2, C3)


def kernel(original_xyzs, dirs, static_table, tableA, tableB, table2A, table2B,
           W1, W2, C1, C2, C3):
    xs = original_xyzs[:, 0]
    ys = original_xyzs[:, 1]
    zs = original_xyzs[:, 2]
    t0 = original_xyzs[0, 3]

    prev1 = 1.0 - (t0 * 16.0 - 8.0)
    nxt1 = 1.0 - prev1
    s1 = prev1 + nxt1
    prev2 = 1.0 - (t0 * 20.0 - 10.0)
    nxt2 = 1.0 - prev2
    s2 = prev2 + nxt2
    par = jnp.concatenate([
        jnp.stack([prev1 / s1, nxt1 / s1, prev2 / s2, nxt2 / s2]),
        jnp.zeros((12,), jnp.float32),
    ])
    resv = jnp.asarray(_RES, dtype=jnp.float32)

    tbl = jnp.concatenate(
        [static_table, tableA, tableB, table2A, table2B,
         jnp.zeros((_L, _T, _D - 5 * _F), jnp.float32)],
        axis=2,
    ).reshape(_L * _T, _D)

    sc_flat = _sc_encode(xs, ys, zs, tbl, resv, par)
    sc_out = sc_flat.reshape(_NPTS, _OC)

    sig2d, color = _tc_mlp(sc_out, dirs, W1, W2, C1, C2, C3)
    sigma = sig2d.reshape(_NPTS)
    af1 = sc_out[:, 96:104]
    af2 = sc_out[:, 104:112]
    return (sigma, color, af1, af2)
